# baseline (speedup 1.0000x reference)
import functools

import jax
import jax.numpy as jnp
from jax import lax
from jax.experimental import pallas as pl
from jax.experimental.pallas import tpu as pltpu
from jax.experimental.pallas import tpu_sc as plsc

_CHUNK = 128


@functools.cache
def _make_kernel(B, D):
    info = plsc.get_sparse_core_info()
    NC, NS = info.num_cores, info.num_subcores
    NW = NC * NS
    assert B % (8 * NW) == 0
    b_per_w = B // NW
    assert b_per_w % _CHUNK == 0
    n_chunks = b_per_w // _CHUNK
    mesh = plsc.VectorSubcoreMesh(core_axis_name="c", subcore_axis_name="s")

    @functools.partial(
        pl.kernel,
        mesh=mesh,
        compiler_params=pltpu.CompilerParams(use_tc_tiling_on_sc=False),
        out_type=jax.ShapeDtypeStruct((2 * B, D), jnp.float32),
        scratch_types=[
            pltpu.VMEM((b_per_w,), jnp.int32),
            pltpu.VMEM((b_per_w,), jnp.int32),
            pltpu.VMEM((n_chunks, _CHUNK), jnp.int32),
            pltpu.VMEM((n_chunks, _CHUNK), jnp.int32),
            pltpu.VMEM((b_per_w, D), jnp.float32),
            pltpu.VMEM((b_per_w, D), jnp.float32),
            pltpu.SemaphoreType.DMA,
            pltpu.SemaphoreType.DMA,
        ],
    )
    def k(rid_hbm, jid_hbm, rtab_hbm, jtab_hbm, oir_hbm, oij_hbm, out_hbm,
          ridx_v, jidx_v, oir_v, oij_v, rrows_v, jrows_v, gsem, ssem):
        wid = lax.axis_index("s") * NC + lax.axis_index("c")
        base = wid * b_per_w
        pltpu.sync_copy(rid_hbm.at[pl.ds(base, b_per_w)], ridx_v)
        pltpu.sync_copy(jid_hbm.at[pl.ds(base, b_per_w)], jidx_v)
        pltpu.sync_copy(oir_hbm.at[wid], oir_v)
        pltpu.sync_copy(oij_hbm.at[wid], oij_v)
        gathers = []
        for c in range(n_chunks):
            sl = pl.ds(c * _CHUNK, _CHUNK)
            gathers.append(pltpu.async_copy(
                rtab_hbm.at[ridx_v.at[sl]], rrows_v.at[sl, :], gsem))
            gathers.append(pltpu.async_copy(
                jtab_hbm.at[jidx_v.at[sl]], jrows_v.at[sl, :], gsem))
        for g in gathers:
            g.wait()
        scatters = []
        for c in range(n_chunks):
            sl = pl.ds(c * _CHUNK, _CHUNK)
            scatters.append(pltpu.async_copy(
                rrows_v.at[sl, :], out_hbm.at[oir_v.at[c]], ssem))
            scatters.append(pltpu.async_copy(
                jrows_v.at[sl, :], out_hbm.at[oij_v.at[c]], ssem))
        for s in scatters:
            s.wait()

    return k


def kernel(road_ids, junction_ids, road_table, junction_table):
    B = road_ids.shape[0]
    D = road_table.shape[1]
    info = plsc.get_sparse_core_info()
    NW = info.num_cores * info.num_subcores
    b_per_w = B // NW
    n_chunks = b_per_w // _CHUNK
    rows = 2 * jnp.arange(B, dtype=jnp.int32)
    oir = rows.reshape(NW, n_chunks, _CHUNK)
    oij = (rows + 1).reshape(NW, n_chunks, _CHUNK)
    k = _make_kernel(B, D)
    out = k(road_ids.astype(jnp.int32), junction_ids.astype(jnp.int32),
            road_table, junction_table, oir, oij)
    return out.reshape(B, 2 * D)


# speedup vs baseline: 1.0012x; 1.0012x over previous
import functools

import jax
import jax.numpy as jnp
from jax import lax
from jax.experimental import pallas as pl
from jax.experimental.pallas import tpu as pltpu
from jax.experimental.pallas import tpu_sc as plsc

_CHUNK = 128


@functools.cache
def _make_kernel(B, D):
    info = plsc.get_sparse_core_info()
    NC, NS = info.num_cores, info.num_subcores
    NW = NC * NS
    assert B % (8 * NW) == 0
    b_per_w = B // NW
    assert b_per_w % _CHUNK == 0
    n_chunks = b_per_w // _CHUNK
    mesh = plsc.VectorSubcoreMesh(core_axis_name="c", subcore_axis_name="s")

    @functools.partial(
        pl.kernel,
        mesh=mesh,
        compiler_params=pltpu.CompilerParams(use_tc_tiling_on_sc=False),
        out_type=jax.ShapeDtypeStruct((B, 2 * D), jnp.float32),
        scratch_types=[
            pltpu.VMEM((b_per_w,), jnp.int32),
            pltpu.VMEM((b_per_w,), jnp.int32),
            pltpu.VMEM((b_per_w, D), jnp.float32),
            pltpu.VMEM((b_per_w, D), jnp.float32),
            pltpu.SemaphoreType.DMA,
        ],
    )
    def k(rid_hbm, jid_hbm, rtab_hbm, jtab_hbm, out_hbm,
          ridx_v, jidx_v, rrows_v, jrows_v, gsem):
        wid = lax.axis_index("s") * NC + lax.axis_index("c")
        base = wid * b_per_w
        pltpu.sync_copy(rid_hbm.at[pl.ds(base, b_per_w)], ridx_v)
        pltpu.sync_copy(jid_hbm.at[pl.ds(base, b_per_w)], jidx_v)
        gathers = []
        for c in range(n_chunks):
            sl = pl.ds(c * _CHUNK, _CHUNK)
            gathers.append(pltpu.async_copy(
                rtab_hbm.at[ridx_v.at[sl]], rrows_v.at[sl, :], gsem))
            gathers.append(pltpu.async_copy(
                jtab_hbm.at[jidx_v.at[sl]], jrows_v.at[sl, :], gsem))
        for g in gathers:
            g.wait()
        pltpu.sync_copy(rrows_v, out_hbm.at[pl.ds(base, b_per_w), pl.ds(0, D)])
        pltpu.sync_copy(jrows_v, out_hbm.at[pl.ds(base, b_per_w), pl.ds(D, D)])

    return k


def kernel(road_ids, junction_ids, road_table, junction_table):
    B = road_ids.shape[0]
    D = road_table.shape[1]
    k = _make_kernel(B, D)
    return k(road_ids.astype(jnp.int32), junction_ids.astype(jnp.int32),
             road_table, junction_table)


# speedup vs baseline: 2.1569x; 2.1543x over previous
import functools

import jax
import jax.numpy as jnp
from jax import lax
from jax.experimental import pallas as pl
from jax.experimental.pallas import tpu as pltpu
from jax.experimental.pallas import tpu_sc as plsc

_CHUNK = 128
_GRP = 16
_RING = 4


@functools.cache
def _make_road_kernel(B, D, V):
    info = plsc.get_sparse_core_info()
    NC, NS = info.num_cores, info.num_subcores
    NW = NC * NS
    b_per_w = B // NW
    n_grp = b_per_w // _GRP
    mesh = plsc.VectorSubcoreMesh(core_axis_name="c", subcore_axis_name="s")

    @functools.partial(
        pl.kernel,
        mesh=mesh,
        compiler_params=pltpu.CompilerParams(
            use_tc_tiling_on_sc=True, needs_layout_passes=False),
        out_type=jax.ShapeDtypeStruct((B, 2 * D), jnp.float32),
        scratch_types=[
            pltpu.VMEM((b_per_w,), jnp.int32),
            pltpu.VMEM((_RING, D, 128), jnp.float32),
            pltpu.VMEM((b_per_w, 2 * D), jnp.float32),
            [pltpu.SemaphoreType.DMA] * _RING,
        ],
    )
    def k(rid_hbm, rtabT_hbm, out_hbm, ridx_v, ring_v, blk_v, sems):
        wid = lax.axis_index("s") * NC + lax.axis_index("c")
        base = wid * b_per_w
        pltpu.sync_copy(rid_hbm.at[pl.ds(base, b_per_w)], ridx_v)

        def start(rid, slot):
            tc = pl.multiple_of((rid >> 7) << 7, 128)
            return pltpu.async_copy(
                rtabT_hbm.at[:, pl.ds(tc, 128)], ring_v.at[slot], sems[slot])

        def extract(rid, slot, row):
            c = rid & 127
            cidx = jnp.full((16,), c, jnp.int32)
            for g in range(D // 16):
                fidx = lax.iota(jnp.int32, 16) + g * 16
                vals = plsc.load_gather(ring_v.at[slot], [fidx, cidx])
                blk_v[row, pl.ds(g * 16, 16)] = vals

        def grp_body(ch, carry):
            idv = ridx_v[pl.ds(ch * _GRP, _GRP)]
            copies = []
            for j in range(_RING):
                copies.append(start(idv[j], j))
            for j in range(_GRP):
                copies[j].wait()
                if j + _RING < _GRP:
                    copies.append(start(idv[j + _RING], (j + _RING) % _RING))
                extract(idv[j], j % _RING, ch * _GRP + j)
            return carry

        lax.fori_loop(0, n_grp, grp_body, 0)
        pltpu.sync_copy(blk_v, out_hbm.at[pl.ds(base, b_per_w), :])

    return k


@functools.cache
def _make_junction_kernel(B, D):
    info = plsc.get_sparse_core_info()
    NC, NS = info.num_cores, info.num_subcores
    NW = NC * NS
    b_per_w = B // NW
    n_chunks = b_per_w // _CHUNK
    mesh = plsc.VectorSubcoreMesh(core_axis_name="c", subcore_axis_name="s")

    @functools.partial(
        pl.kernel,
        mesh=mesh,
        compiler_params=pltpu.CompilerParams(use_tc_tiling_on_sc=False),
        out_type=jax.ShapeDtypeStruct((B, 2 * D), jnp.float32),
        scratch_types=[
            pltpu.VMEM((b_per_w,), jnp.int32),
            pltpu.VMEM((b_per_w, D), jnp.float32),
            pltpu.VMEM((b_per_w, D), jnp.float32),
            pltpu.SemaphoreType.DMA,
        ],
    )
    def k(jid_hbm, jtab_hbm, road_hbm, out_hbm, jidx_v, jrows_v, rrows_v, gsem):
        wid = lax.axis_index("s") * NC + lax.axis_index("c")
        base = wid * b_per_w
        pltpu.sync_copy(jid_hbm.at[pl.ds(base, b_per_w)], jidx_v)
        rcopy = pltpu.async_copy(
            road_hbm.at[pl.ds(base, b_per_w), pl.ds(0, D)], rrows_v, gsem)
        gathers = []
        for c in range(n_chunks):
            sl = pl.ds(c * _CHUNK, _CHUNK)
            gathers.append(pltpu.async_copy(
                jtab_hbm.at[jidx_v.at[sl]], jrows_v.at[sl, :], gsem))
        rcopy.wait()
        for g in gathers:
            g.wait()
        pltpu.sync_copy(rrows_v, out_hbm.at[pl.ds(base, b_per_w), pl.ds(0, D)])
        pltpu.sync_copy(jrows_v, out_hbm.at[pl.ds(base, b_per_w), pl.ds(D, D)])

    return k


def kernel(road_ids, junction_ids, road_table, junction_table):
    B = road_ids.shape[0]
    D = road_table.shape[1]
    ka = _make_road_kernel(B, D, road_table.shape[0])
    kb = _make_junction_kernel(B, D)
    road_half = ka(road_ids.astype(jnp.int32), road_table.T)
    return kb(junction_ids.astype(jnp.int32), junction_table, road_half)


# speedup vs baseline: 2.4062x; 1.1156x over previous
import functools

import jax
import jax.numpy as jnp
from jax import lax
from jax.experimental import pallas as pl
from jax.experimental.pallas import tpu as pltpu
from jax.experimental.pallas import tpu_sc as plsc

_CHUNK = 128
_GRP = 64
_RING = 6


@functools.cache
def _make_road_kernel(B, D, V):
    info = plsc.get_sparse_core_info()
    NC, NS = info.num_cores, info.num_subcores
    NW = NC * NS
    b_per_w = B // NW
    n_grp = b_per_w // _GRP
    mesh = plsc.VectorSubcoreMesh(core_axis_name="c", subcore_axis_name="s")

    @functools.partial(
        pl.kernel,
        mesh=mesh,
        compiler_params=pltpu.CompilerParams(
            use_tc_tiling_on_sc=True, needs_layout_passes=False),
        out_type=jax.ShapeDtypeStruct((B, 2 * D), jnp.float32),
        scratch_types=[
            pltpu.VMEM((b_per_w,), jnp.int32),
            pltpu.VMEM((_RING, D, 128), jnp.float32),
            pltpu.VMEM((b_per_w, 2 * D), jnp.float32),
            [pltpu.SemaphoreType.DMA] * _RING,
        ],
    )
    def k(rid_hbm, rtabT_hbm, out_hbm, ridx_v, ring_v, blk_v, sems):
        wid = lax.axis_index("s") * NC + lax.axis_index("c")
        base = wid * b_per_w
        pltpu.sync_copy(rid_hbm.at[pl.ds(base, b_per_w)], ridx_v)

        def start(rid, slot):
            tc = pl.multiple_of((rid >> 7) << 7, 128)
            return pltpu.async_copy(
                rtabT_hbm.at[:, pl.ds(tc, 128)], ring_v.at[slot], sems[slot])

        def extract(rid, slot, row):
            c = rid & 127
            cidx = jnp.full((16,), c, jnp.int32)
            for g in range(D // 16):
                fidx = lax.iota(jnp.int32, 16) + g * 16
                vals = plsc.load_gather(ring_v.at[slot], [fidx, cidx])
                blk_v[row, pl.ds(g * 16, 16)] = vals

        def grp_body(ch, carry):
            idvs = [ridx_v[pl.ds(ch * _GRP + g * 16, 16)]
                    for g in range(_GRP // 16)]
            ids = [idvs[j // 16][j % 16] for j in range(_GRP)]
            copies = []
            for j in range(_RING):
                copies.append(start(ids[j], j))
            for j in range(_GRP):
                copies[j].wait()
                if j + _RING < _GRP:
                    copies.append(start(ids[j + _RING], (j + _RING) % _RING))
                extract(ids[j], j % _RING, ch * _GRP + j)
            return carry

        lax.fori_loop(0, n_grp, grp_body, 0)
        pltpu.sync_copy(blk_v, out_hbm.at[pl.ds(base, b_per_w), :])

    return k


@functools.cache
def _make_junction_kernel(B, D):
    info = plsc.get_sparse_core_info()
    NC, NS = info.num_cores, info.num_subcores
    NW = NC * NS
    b_per_w = B // NW
    n_chunks = b_per_w // _CHUNK
    mesh = plsc.VectorSubcoreMesh(core_axis_name="c", subcore_axis_name="s")

    @functools.partial(
        pl.kernel,
        mesh=mesh,
        compiler_params=pltpu.CompilerParams(use_tc_tiling_on_sc=False),
        out_type=jax.ShapeDtypeStruct((B, 2 * D), jnp.float32),
        scratch_types=[
            pltpu.VMEM((b_per_w,), jnp.int32),
            pltpu.VMEM((b_per_w, D), jnp.float32),
            pltpu.VMEM((b_per_w, D), jnp.float32),
            pltpu.SemaphoreType.DMA,
        ],
    )
    def k(jid_hbm, jtab_hbm, road_hbm, out_hbm, jidx_v, jrows_v, rrows_v, gsem):
        wid = lax.axis_index("s") * NC + lax.axis_index("c")
        base = wid * b_per_w
        pltpu.sync_copy(jid_hbm.at[pl.ds(base, b_per_w)], jidx_v)
        rcopy = pltpu.async_copy(
            road_hbm.at[pl.ds(base, b_per_w), pl.ds(0, D)], rrows_v, gsem)
        gathers = []
        for c in range(n_chunks):
            sl = pl.ds(c * _CHUNK, _CHUNK)
            gathers.append(pltpu.async_copy(
                jtab_hbm.at[jidx_v.at[sl]], jrows_v.at[sl, :], gsem))
        rcopy.wait()
        for g in gathers:
            g.wait()
        pltpu.sync_copy(rrows_v, out_hbm.at[pl.ds(base, b_per_w), pl.ds(0, D)])
        pltpu.sync_copy(jrows_v, out_hbm.at[pl.ds(base, b_per_w), pl.ds(D, D)])

    return k


def kernel(road_ids, junction_ids, road_table, junction_table):
    B = road_ids.shape[0]
    D = road_table.shape[1]
    ka = _make_road_kernel(B, D, road_table.shape[0])
    kb = _make_junction_kernel(B, D)
    road_half = ka(road_ids.astype(jnp.int32), road_table.T)
    return kb(junction_ids.astype(jnp.int32), junction_table, road_half)


# speedup vs baseline: 3.2737x; 1.3605x over previous
import functools

import jax
import jax.numpy as jnp
from jax import lax
from jax.experimental import pallas as pl
from jax.experimental.pallas import tpu as pltpu
from jax.experimental.pallas import tpu_sc as plsc

_CHUNK = 128
_GRP = 128
_RING = 8
_KMAX = 6


@functools.cache
def _make_road_kernel(B, D, V):
    info = plsc.get_sparse_core_info()
    NC, NS = info.num_cores, info.num_subcores
    NW = NC * NS
    b_per_w = B // NW
    n_grp = b_per_w // _GRP
    mesh = plsc.VectorSubcoreMesh(core_axis_name="c", subcore_axis_name="s")

    @functools.partial(
        pl.kernel,
        mesh=mesh,
        compiler_params=pltpu.CompilerParams(
            use_tc_tiling_on_sc=True, needs_layout_passes=False),
        out_type=jax.ShapeDtypeStruct((B, 2 * D), jnp.float32),
        scratch_types=[
            pltpu.VMEM((b_per_w,), jnp.int32),
            pltpu.VMEM((b_per_w,), jnp.int32),
            pltpu.VMEM((_RING, D, 128), jnp.float32),
            pltpu.VMEM((_GRP, 2 * D), jnp.float32),
            [pltpu.SemaphoreType.DMA] * _RING,
        ],
    )
    def k(sid_hbm, kpos_hbm, rtabT_hbm, out_hbm, sidx_v, kpos_v, ring_v,
          blk_v, sems):
        wid = lax.axis_index("s") * NC + lax.axis_index("c")
        base = wid * b_per_w
        pltpu.sync_copy(sid_hbm.at[pl.ds(base, b_per_w)], sidx_v)
        pltpu.sync_copy(kpos_hbm.at[pl.ds(base, b_per_w)], kpos_v)

        def start(rid, kpos, slot):
            tc = pl.multiple_of((rid >> 7) << 7, 128)
            @pl.when(kpos == 0)
            def _():
                pltpu.async_copy(
                    rtabT_hbm.at[:, pl.ds(tc, 128)], ring_v.at[slot],
                    sems[slot])

        def wait(kpos, slot):
            @pl.when(kpos == 0)
            def _():
                pltpu.make_async_copy(
                    rtabT_hbm.at[:, pl.ds(0, 128)], ring_v.at[slot],
                    sems[slot]).wait()

        def extract(rid, kpos, j, row):
            c = rid & 127
            slot = (jnp.int32(j) - kpos) % _RING
            cidx = jnp.full((16,), c, jnp.int32)
            sidxv = jnp.full((16,), slot, jnp.int32)
            for g in range(D // 16):
                fidx = lax.iota(jnp.int32, 16) + g * 16
                vals = plsc.load_gather(ring_v, [sidxv, fidx, cidx])
                blk_v[row, pl.ds(g * 16, 16)] = vals

        def grp_body(ch, carry):
            gb = ch * _GRP
            idvs = [sidx_v[pl.ds(gb + g * 16, 16)] for g in range(_GRP // 16)]
            kvs = [kpos_v[pl.ds(gb + g * 16, 16)] for g in range(_GRP // 16)]
            ids = [idvs[j // 16][j % 16] for j in range(_GRP)]
            kps = [kvs[j // 16][j % 16] for j in range(_GRP)]
            for j in range(_RING):
                start(ids[j], kps[j], j)
            for j in range(_GRP):
                wait(kps[j], j % _RING)
                extract(ids[j], kps[j], j, j)
                if j + _RING < _GRP:
                    start(ids[j + _RING], kps[j + _RING], (j + _RING) % _RING)
            out_off = pl.multiple_of(base + gb, _GRP)
            pltpu.sync_copy(blk_v, out_hbm.at[pl.ds(out_off, _GRP), :])
            return carry

        lax.fori_loop(0, n_grp, grp_body, 0)

    return k


@functools.cache
def _make_junction_kernel(B, D):
    info = plsc.get_sparse_core_info()
    NC, NS = info.num_cores, info.num_subcores
    NW = NC * NS
    b_per_w = B // NW
    n_chunks = b_per_w // _CHUNK
    mesh = plsc.VectorSubcoreMesh(core_axis_name="c", subcore_axis_name="s")

    @functools.partial(
        pl.kernel,
        mesh=mesh,
        compiler_params=pltpu.CompilerParams(use_tc_tiling_on_sc=False),
        out_type=jax.ShapeDtypeStruct((B, 2 * D), jnp.float32),
        scratch_types=[
            pltpu.VMEM((b_per_w,), jnp.int32),
            pltpu.VMEM((b_per_w,), jnp.int32),
            pltpu.VMEM((b_per_w, D), jnp.float32),
            pltpu.VMEM((b_per_w, 2 * D), jnp.float32),
            pltpu.SemaphoreType.DMA,
        ],
    )
    def k(jid_hbm, inv_hbm, jtab_hbm, road_hbm, out_hbm,
          jidx_v, iidx_v, jrows_v, rrows_v, gsem):
        wid = lax.axis_index("s") * NC + lax.axis_index("c")
        base = wid * b_per_w
        pltpu.sync_copy(jid_hbm.at[pl.ds(base, b_per_w)], jidx_v)
        pltpu.sync_copy(inv_hbm.at[pl.ds(base, b_per_w)], iidx_v)
        gathers = []
        for c in range(n_chunks):
            sl = pl.ds(c * _CHUNK, _CHUNK)
            gathers.append(pltpu.async_copy(
                road_hbm.at[iidx_v.at[sl]], rrows_v.at[sl, :], gsem))
            gathers.append(pltpu.async_copy(
                jtab_hbm.at[jidx_v.at[sl]], jrows_v.at[sl, :], gsem))
        for g in gathers:
            g.wait()
        pltpu.sync_copy(rrows_v.at[:, pl.ds(0, D)],
                        out_hbm.at[pl.ds(base, b_per_w), pl.ds(0, D)])
        pltpu.sync_copy(jrows_v, out_hbm.at[pl.ds(base, b_per_w), pl.ds(D, D)])

    return k


def kernel(road_ids, junction_ids, road_table, junction_table):
    B = road_ids.shape[0]
    D = road_table.shape[1]
    rid32 = road_ids.astype(jnp.int32)
    perm = jnp.argsort(rid32)
    sids = rid32[perm]
    inv = jnp.zeros((B,), jnp.int32).at[perm].set(
        jnp.arange(B, dtype=jnp.int32))
    cols = sids >> 7
    idx = jnp.arange(B, dtype=jnp.int32)
    new_run = jnp.concatenate(
        [jnp.ones((1,), jnp.bool_), cols[1:] != cols[:-1]])
    seg_start = lax.associative_scan(jnp.maximum, jnp.where(new_run, idx, 0))
    info = plsc.get_sparse_core_info()
    b_per_w = B // (info.num_cores * info.num_subcores)
    kpos = jnp.minimum((idx - seg_start) % (_KMAX + 1), idx % b_per_w)

    ka = _make_road_kernel(B, D, road_table.shape[0])
    kb = _make_junction_kernel(B, D)
    road_sorted = ka(sids, kpos, road_table.T)
    return kb(junction_ids.astype(jnp.int32), inv, junction_table,
              road_sorted)


# speedup vs baseline: 3.3832x; 1.0335x over previous
import functools

import jax
import jax.numpy as jnp
from jax import lax
from jax.experimental import pallas as pl
from jax.experimental.pallas import tpu as pltpu
from jax.experimental.pallas import tpu_sc as plsc

_CHUNK = 128
_GRP = 128
_RING = 8
_KMAX = 6


@functools.cache
def _make_road_kernel(B, D, V):
    info = plsc.get_sparse_core_info()
    NC, NS = info.num_cores, info.num_subcores
    NW = NC * NS
    b_per_w = B // NW
    n_grp = b_per_w // _GRP
    mesh = plsc.VectorSubcoreMesh(core_axis_name="c", subcore_axis_name="s")

    @functools.partial(
        pl.kernel,
        mesh=mesh,
        compiler_params=pltpu.CompilerParams(
            use_tc_tiling_on_sc=True, needs_layout_passes=False),
        out_type=jax.ShapeDtypeStruct((B, 2 * D), jnp.float32),
        scratch_types=[
            pltpu.VMEM((b_per_w,), jnp.int32),
            pltpu.VMEM((b_per_w,), jnp.int32),
            pltpu.VMEM((_RING, D, 128), jnp.float32),
            pltpu.VMEM((_GRP, 2 * D), jnp.float32),
            [pltpu.SemaphoreType.DMA] * _RING,
        ],
    )
    def k(sid_hbm, kpos_hbm, rtabT_hbm, out_hbm, sidx_v, kpos_v, ring_v,
          blk_v, sems):
        wid = lax.axis_index("s") * NC + lax.axis_index("c")
        base = wid * b_per_w
        pltpu.sync_copy(sid_hbm.at[pl.ds(base, b_per_w)], sidx_v)
        pltpu.sync_copy(kpos_hbm.at[pl.ds(base, b_per_w)], kpos_v)

        def start(rid, kpos, slot):
            tc = pl.multiple_of((rid >> 7) << 7, 128)
            @pl.when(kpos == 0)
            def _():
                pltpu.async_copy(
                    rtabT_hbm.at[:, pl.ds(tc, 128)], ring_v.at[slot],
                    sems[slot])

        def wait(kpos, slot):
            @pl.when(kpos == 0)
            def _():
                pltpu.make_async_copy(
                    rtabT_hbm.at[:, pl.ds(0, 128)], ring_v.at[slot],
                    sems[slot]).wait()

        def extract(rid, kpos, j, row):
            c = rid & 127
            slot = (jnp.int32(j) - kpos) % _RING
            cidx = jnp.full((16,), c, jnp.int32)
            sidxv = jnp.full((16,), slot, jnp.int32)
            for g in range(D // 16):
                fidx = lax.iota(jnp.int32, 16) + g * 16
                vals = plsc.load_gather(ring_v, [sidxv, fidx, cidx])
                blk_v[row, pl.ds(g * 16, 16)] = vals

        def grp_body(ch, carry):
            gb = ch * _GRP
            idvs = [sidx_v[pl.ds(gb + g * 16, 16)] for g in range(_GRP // 16)]
            kvs = [kpos_v[pl.ds(gb + g * 16, 16)] for g in range(_GRP // 16)]
            ids = [idvs[j // 16][j % 16] for j in range(_GRP)]
            kps = [kvs[j // 16][j % 16] for j in range(_GRP)]
            for j in range(_RING):
                start(ids[j], kps[j], j)
            for j in range(_GRP):
                wait(kps[j], j % _RING)
                extract(ids[j], kps[j], j, j)
                if j + _RING < _GRP:
                    start(ids[j + _RING], kps[j + _RING], (j + _RING) % _RING)
            out_off = pl.multiple_of(base + gb, _GRP)
            pltpu.sync_copy(blk_v, out_hbm.at[pl.ds(out_off, _GRP), :])
            return carry

        lax.fori_loop(0, n_grp, grp_body, 0)

    return k


@functools.cache
def _make_junction_kernel(B, D):
    info = plsc.get_sparse_core_info()
    NC, NS = info.num_cores, info.num_subcores
    NW = NC * NS
    b_per_w = B // NW
    n_chunks = b_per_w // _CHUNK
    mesh = plsc.VectorSubcoreMesh(core_axis_name="c", subcore_axis_name="s")

    @functools.partial(
        pl.kernel,
        mesh=mesh,
        compiler_params=pltpu.CompilerParams(use_tc_tiling_on_sc=False),
        out_type=jax.ShapeDtypeStruct((B, 2 * D), jnp.float32),
        scratch_types=[
            pltpu.VMEM((b_per_w,), jnp.int32),
            pltpu.VMEM((b_per_w,), jnp.int32),
            pltpu.VMEM((b_per_w, D), jnp.float32),
            pltpu.VMEM((b_per_w, 2 * D), jnp.float32),
            pltpu.SemaphoreType.DMA,
        ],
    )
    def k(jid_hbm, inv_hbm, jtab_hbm, road_hbm, out_hbm,
          jidx_v, iidx_v, jrows_v, rrows_v, gsem):
        wid = lax.axis_index("s") * NC + lax.axis_index("c")
        base = wid * b_per_w
        pltpu.sync_copy(jid_hbm.at[pl.ds(base, b_per_w)], jidx_v)
        pltpu.sync_copy(inv_hbm.at[pl.ds(base, b_per_w)], iidx_v)
        gathers = []
        for c in range(n_chunks):
            sl = pl.ds(c * _CHUNK, _CHUNK)
            gathers.append(pltpu.async_copy(
                road_hbm.at[iidx_v.at[sl]], rrows_v.at[sl, :], gsem))
            gathers.append(pltpu.async_copy(
                jtab_hbm.at[jidx_v.at[sl]], jrows_v.at[sl, :], gsem))
        for g in gathers:
            g.wait()
        pltpu.sync_copy(rrows_v.at[:, pl.ds(0, D)],
                        out_hbm.at[pl.ds(base, b_per_w), pl.ds(0, D)])
        pltpu.sync_copy(jrows_v, out_hbm.at[pl.ds(base, b_per_w), pl.ds(D, D)])

    return k


def kernel(road_ids, junction_ids, road_table, junction_table):
    B = road_ids.shape[0]
    D = road_table.shape[1]
    rid32 = road_ids.astype(jnp.int32)
    iota = jnp.arange(B, dtype=jnp.int32)
    sids, perm = lax.sort_key_val(rid32, iota)
    inv = jnp.zeros((B,), jnp.int32).at[perm].set(iota)
    cols = sids >> 7
    idx = jnp.arange(B, dtype=jnp.int32)
    new_run = jnp.concatenate(
        [jnp.ones((1,), jnp.bool_), cols[1:] != cols[:-1]])
    seg_start = lax.associative_scan(jnp.maximum, jnp.where(new_run, idx, 0))
    info = plsc.get_sparse_core_info()
    b_per_w = B // (info.num_cores * info.num_subcores)
    kpos = jnp.minimum((idx - seg_start) % (_KMAX + 1), idx % b_per_w)

    ka = _make_road_kernel(B, D, road_table.shape[0])
    kb = _make_junction_kernel(B, D)
    road_sorted = ka(sids, kpos, road_table.T)
    return kb(junction_ids.astype(jnp.int32), inv, junction_table,
              road_sorted)
